# baseline (speedup 1.0000x reference)
import functools

import jax
import jax.numpy as jnp
from jax import lax
from jax.experimental import pallas as pl
from jax.experimental.pallas import tpu as pltpu
from jax.experimental.pallas import tpu_sc as plsc

_ROWS, _COLS = 16384, 4096
_N = _ROWS * _COLS
_NC, _NS, _L = 2, 16, 16
_NW = _NC * _NS
_PER_W = _N // _NW
_CH = 65536
_N_CHUNKS = _PER_W // _CH


def _make_sc_kernel():
    mesh = plsc.VectorSubcoreMesh(core_axis_name="c", subcore_axis_name="s")

    @functools.partial(
        pl.kernel,
        mesh=mesh,
        out_type=jax.ShapeDtypeStruct((_N,), jnp.float32),
        scratch_types=[pltpu.VMEM((_CH,), jnp.float32)],
    )
    def k(in_hbm, out_hbm, buf):
        wid = lax.axis_index("s") * _NC + lax.axis_index("c")
        base = wid * _PER_W

        def chunk_body(j, carry):
            off = base + j * _CH
            pltpu.sync_copy(in_hbm.at[pl.ds(off, _CH)], buf)

            ones = jnp.full((_L,), 1.0, jnp.float32)
            zeros = jnp.zeros((_L,), jnp.float32)

            def vec_body(i, c2):
                v = buf[pl.ds(i * _L, _L)]
                buf[pl.ds(i * _L, _L)] = jnp.where(v != 0.0, ones, zeros)
                return c2

            lax.fori_loop(0, _CH // _L, vec_body, 0)
            pltpu.sync_copy(buf, out_hbm.at[pl.ds(off, _CH)])
            return carry

        lax.fori_loop(0, _N_CHUNKS, chunk_body, 0)

    return k


_sc_kernel = _make_sc_kernel()


def kernel(mask):
    flat = mask.reshape(_N)
    out = _sc_kernel(flat)
    return out.reshape(_ROWS, _COLS)


# speedup vs baseline: 2.4134x; 2.4134x over previous
import functools

import jax
import jax.numpy as jnp
from jax import lax
from jax.experimental import pallas as pl
from jax.experimental.pallas import tpu as pltpu
from jax.experimental.pallas import tpu_sc as plsc

_ROWS, _COLS = 16384, 4096
_SC_ROWS = 1024
_TC_ROWS = _ROWS - _SC_ROWS
_NC, _NS, _L = 2, 16, 16
_NW = _NC * _NS
_SC_N = _SC_ROWS * _COLS
_PER_W = _SC_N // _NW
_CH = 65536
_N_CHUNKS = _PER_W // _CH
_SC_BASE = _TC_ROWS * _COLS


def _tc_body(mask_ref, out_ref):
    out_ref[...] = (mask_ref[...] != 0.0).astype(jnp.float32)


def _make_sc_kernel():
    mesh = plsc.VectorSubcoreMesh(core_axis_name="c", subcore_axis_name="s")

    @functools.partial(
        pl.kernel,
        mesh=mesh,
        out_type=jax.ShapeDtypeStruct((_SC_N,), jnp.float32),
        scratch_types=[pltpu.VMEM((_CH,), jnp.float32)],
    )
    def k(in_hbm, out_hbm, buf):
        wid = lax.axis_index("s") * _NC + lax.axis_index("c")
        base = wid * _PER_W

        def chunk_body(j, carry):
            off = base + j * _CH
            pltpu.sync_copy(in_hbm.at[pl.ds(_SC_BASE + off, _CH)], buf)

            ones = jnp.full((_L,), 1.0, jnp.float32)
            zeros = jnp.zeros((_L,), jnp.float32)

            def vec_body(i, c2):
                v = buf[pl.ds(i * _L, _L)]
                buf[pl.ds(i * _L, _L)] = jnp.where(v != 0.0, ones, zeros)
                return c2

            lax.fori_loop(0, _CH // _L, vec_body, 0)
            pltpu.sync_copy(buf, out_hbm.at[pl.ds(off, _CH)])
            return carry

        lax.fori_loop(0, _N_CHUNKS, chunk_body, 0)

    return k


_sc_kernel = _make_sc_kernel()

_TC_BLOCK = 512


def _tc_kernel(mask):
    return pl.pallas_call(
        _tc_body,
        grid=(_TC_ROWS // _TC_BLOCK,),
        in_specs=[pl.BlockSpec((_TC_BLOCK, _COLS), lambda i: (i, 0))],
        out_specs=pl.BlockSpec((_TC_BLOCK, _COLS), lambda i: (i, 0)),
        out_shape=jax.ShapeDtypeStruct((_TC_ROWS, _COLS), jnp.float32),
    )(mask)


def kernel(mask):
    flat = mask.reshape(_ROWS * _COLS)
    sc_out = _sc_kernel(flat)
    tc_out = _tc_kernel(mask)
    return jnp.concatenate([tc_out, sc_out.reshape(_SC_ROWS, _COLS)], axis=0)


# speedup vs baseline: 2.4182x; 1.0020x over previous
import functools

import jax
import jax.numpy as jnp
from jax import lax
from jax.experimental import pallas as pl
from jax.experimental.pallas import tpu as pltpu
from jax.experimental.pallas import tpu_sc as plsc

_ROWS, _COLS = 16384, 4096
_SC_ROWS = 1024
_TC_ROWS = _ROWS - _SC_ROWS
_NC, _NS, _L = 2, 16, 16
_NW = _NC * _NS
_SC_N = _SC_ROWS * _COLS
_PER_W = _SC_N // _NW
_CH = 65536
_N_CHUNKS = _PER_W // _CH


def _tc_body(mask_ref, out_ref):
    out_ref[...] = (mask_ref[...] != 0.0).astype(jnp.float32)


def _make_sc_kernel():
    mesh = plsc.VectorSubcoreMesh(core_axis_name="c", subcore_axis_name="s")

    @functools.partial(
        pl.kernel,
        mesh=mesh,
        out_type=jax.ShapeDtypeStruct((_SC_N,), jnp.float32),
        scratch_types=[pltpu.VMEM((_CH,), jnp.float32)],
    )
    def k(in_hbm, out_hbm, buf):
        wid = lax.axis_index("s") * _NC + lax.axis_index("c")
        base = wid * _PER_W

        def chunk_body(j, carry):
            off = base + j * _CH
            pltpu.sync_copy(in_hbm.at[pl.ds(off, _CH)], buf)

            ones = jnp.full((_L,), 1.0, jnp.float32)
            zeros = jnp.zeros((_L,), jnp.float32)

            def vec_body(i, c2):
                v = buf[pl.ds(i * _L, _L)]
                buf[pl.ds(i * _L, _L)] = jnp.where(v != 0.0, ones, zeros)
                return c2

            lax.fori_loop(0, _CH // _L, vec_body, 0)
            pltpu.sync_copy(buf, out_hbm.at[pl.ds(off, _CH)])
            return carry

        lax.fori_loop(0, _N_CHUNKS, chunk_body, 0)

    return k


_sc_kernel = _make_sc_kernel()

_TC_BLOCK = 512


def _tc_kernel(mask):
    off_blocks = _SC_ROWS // _TC_BLOCK
    return pl.pallas_call(
        _tc_body,
        grid=(_TC_ROWS // _TC_BLOCK,),
        in_specs=[pl.BlockSpec((_TC_BLOCK, _COLS), lambda i: (i + off_blocks, 0))],
        out_specs=pl.BlockSpec((_TC_BLOCK, _COLS), lambda i: (i, 0)),
        out_shape=jax.ShapeDtypeStruct((_TC_ROWS, _COLS), jnp.float32),
    )(mask)


def kernel(mask):
    flat = mask.reshape(_ROWS * _COLS)
    sc_out = _sc_kernel(flat)
    tc_out = _tc_kernel(mask)
    return jnp.concatenate([sc_out.reshape(_SC_ROWS, _COLS), tc_out], axis=0)


# speedup vs baseline: 7.9542x; 3.2893x over previous
import jax
import jax.numpy as jnp
from jax.experimental import pallas as pl


def _mask_kernel(mask_ref, out_ref):
    out_ref[...] = (mask_ref[...] != 0.0).astype(jnp.float32)


def kernel(mask):
    rows, cols = mask.shape
    block_rows = 256
    grid = (rows // block_rows,)
    return pl.pallas_call(
        _mask_kernel,
        grid=grid,
        in_specs=[pl.BlockSpec((block_rows, cols), lambda i: (i, 0))],
        out_specs=pl.BlockSpec((block_rows, cols), lambda i: (i, 0)),
        out_shape=jax.ShapeDtypeStruct((rows, cols), jnp.float32),
    )(mask)


# speedup vs baseline: 8.0621x; 1.0136x over previous
import jax
import jax.numpy as jnp
from jax.experimental import pallas as pl


def _mask_kernel(mask_ref, out_ref):
    out_ref[...] = (mask_ref[...] != 0.0).astype(jnp.float32)


def kernel(mask):
    rows, cols = mask.shape
    block_rows = 512
    grid = (rows // block_rows,)
    return pl.pallas_call(
        _mask_kernel,
        grid=grid,
        in_specs=[pl.BlockSpec((block_rows, cols), lambda i: (i, 0))],
        out_specs=pl.BlockSpec((block_rows, cols), lambda i: (i, 0)),
        out_shape=jax.ShapeDtypeStruct((rows, cols), jnp.float32),
    )(mask)


# speedup vs baseline: 8.0892x; 1.0034x over previous
import jax
import jax.numpy as jnp
from jax.experimental import pallas as pl
from jax.experimental.pallas import tpu as pltpu


def _mask_kernel(mask_ref, out_ref):
    out_ref[...] = (mask_ref[...] != 0.0).astype(jnp.float32)


def kernel(mask):
    rows, cols = mask.shape
    block_rows = 968
    grid = (pl.cdiv(rows, block_rows),)
    return pl.pallas_call(
        _mask_kernel,
        grid=grid,
        in_specs=[pl.BlockSpec((block_rows, cols), lambda i: (i, 0))],
        out_specs=pl.BlockSpec((block_rows, cols), lambda i: (i, 0)),
        out_shape=jax.ShapeDtypeStruct((rows, cols), jnp.float32),
        compiler_params=pltpu.CompilerParams(vmem_limit_bytes=128 * 1024 * 1024),
    )(mask)


# speedup vs baseline: 8.1127x; 1.0029x over previous
import jax
import jax.numpy as jnp
from jax.experimental import pallas as pl
from jax.experimental.pallas import tpu as pltpu


def _mask_kernel(mask_ref, out_ref):
    out_ref[...] = (mask_ref[...] != 0.0).astype(jnp.float32)


def kernel(mask):
    rows, cols = mask.shape
    block_rows = 1016
    grid = (pl.cdiv(rows, block_rows),)
    return pl.pallas_call(
        _mask_kernel,
        grid=grid,
        in_specs=[pl.BlockSpec((block_rows, cols), lambda i: (i, 0))],
        out_specs=pl.BlockSpec((block_rows, cols), lambda i: (i, 0)),
        out_shape=jax.ShapeDtypeStruct((rows, cols), jnp.float32),
        compiler_params=pltpu.CompilerParams(vmem_limit_bytes=128 * 1024 * 1024),
    )(mask)
